# baseline (speedup 1.0000x reference)
import functools

import jax
import jax.numpy as jnp
from jax.experimental import pallas as pl

N = 16384
C = 1000
K = N // 2
ROWS = 4096
GRID = N // ROWS


def _loss_body(pred_ref, tgt_ref, out_ref):
    x = pred_ref[...]
    t = tgt_ref[0, 0, :]
    m = jnp.max(x, axis=1, keepdims=True)
    s = jnp.sum(jnp.exp(x - m), axis=1, keepdims=True)
    lse = m + jnp.log(s)
    col = jax.lax.broadcasted_iota(jnp.int32, x.shape, 1)
    picked = jnp.sum(jnp.where(col == t[:, None], x, 0.0), axis=1,
                     keepdims=True)
    out_ref[0, 0, :] = (lse - picked)[:, 0]


def _select_body(loss_ref, out_ref):
    x = loss_ref[...]
    bits = jax.lax.bitcast_convert_type(x, jnp.int32)

    def body(_, carry):
        lo, hi = carry
        mid = lo + (hi - lo) // 2
        cnt = jnp.sum((bits >= mid).astype(jnp.int32))
        take = cnt >= K
        return jnp.where(take, mid, lo), jnp.where(take, hi, mid)

    lo0 = jnp.int32(0)
    hi0 = jnp.int32(0x7F800000)
    lo, _ = jax.lax.fori_loop(0, 31, body, (lo0, hi0))
    thr = jax.lax.bitcast_convert_type(lo, jnp.float32)
    gt = bits > lo
    cnt_gt = jnp.sum(gt.astype(jnp.int32))
    sum_gt = jnp.sum(jnp.where(gt, x, 0.0))
    total = sum_gt + (K - cnt_gt).astype(jnp.float32) * thr
    out_ref[...] = jnp.reshape(total / jnp.float32(K), (1, 1))


@jax.jit
def kernel(pred, target):
    tgt3 = target.reshape(GRID, 1, ROWS)
    loss = pl.pallas_call(
        _loss_body,
        grid=(GRID,),
        in_specs=[
            pl.BlockSpec((ROWS, C), lambda i: (i, 0)),
            pl.BlockSpec((1, 1, ROWS), lambda i: (i, 0, 0)),
        ],
        out_specs=pl.BlockSpec((1, 1, ROWS), lambda i: (i, 0, 0)),
        out_shape=jax.ShapeDtypeStruct((GRID, 1, ROWS), jnp.float32),
    )(pred, tgt3)

    out = pl.pallas_call(
        _select_body,
        out_shape=jax.ShapeDtypeStruct((1, 1), jnp.float32),
    )(loss)
    return out[0, 0]


# speedup vs baseline: 1.0369x; 1.0369x over previous
import jax
import jax.numpy as jnp
from jax.experimental import pallas as pl
from jax.experimental.pallas import tpu as pltpu

N = 16384
C = 1000
K = N // 2
CH = 512
NCH = N // CH
NBUF = 4


def _body(pred_hbm, tgt_ref, out_ref, buf, loss_ref, sems):
    def start(i):
        pltpu.make_async_copy(
            pred_hbm.at[pl.ds(i * CH, CH), :], buf.at[i % NBUF],
            sems.at[i % NBUF]).start()

    for i in range(NBUF):
        start(i)

    for c in range(NCH):
        pltpu.make_async_copy(
            pred_hbm.at[pl.ds(c * CH, CH), :], buf.at[c % NBUF],
            sems.at[c % NBUF]).wait()
        x = buf[c % NBUF]
        t = tgt_ref[c, 0, :]
        m = jnp.max(x, axis=1, keepdims=True)
        s = jnp.sum(jnp.exp(x - m), axis=1, keepdims=True)
        lse = m + jnp.log(s)
        col = jax.lax.broadcasted_iota(jnp.int32, x.shape, 1)
        picked = jnp.sum(jnp.where(col == t[:, None], x, 0.0), axis=1,
                         keepdims=True)
        loss_ref[c, :] = (lse - picked)[:, 0]
        if c + NBUF < NCH:
            start(c + NBUF)

    x = loss_ref[...]
    bits = jax.lax.bitcast_convert_type(x, jnp.int32)

    def bsearch(_, carry):
        lo, hi = carry
        mid = lo + (hi - lo) // 2
        cnt = jnp.sum((bits >= mid).astype(jnp.int32))
        take = cnt >= K
        return jnp.where(take, mid, lo), jnp.where(take, hi, mid)

    lo, _ = jax.lax.fori_loop(0, 31, bsearch,
                              (jnp.int32(0), jnp.int32(0x7F800000)))
    thr = jax.lax.bitcast_convert_type(lo, jnp.float32)
    gt = bits > lo
    cnt_gt = jnp.sum(gt.astype(jnp.int32))
    sum_gt = jnp.sum(jnp.where(gt, x, 0.0))
    total = sum_gt + (K - cnt_gt).astype(jnp.float32) * thr
    out_ref[...] = jnp.reshape(total / jnp.float32(K), (1, 1))


@jax.jit
def kernel(pred, target):
    tgt3 = target.reshape(NCH, 1, CH)
    out = pl.pallas_call(
        _body,
        in_specs=[
            pl.BlockSpec(memory_space=pltpu.MemorySpace.HBM),
            pl.BlockSpec(memory_space=pltpu.MemorySpace.VMEM),
        ],
        out_specs=pl.BlockSpec(memory_space=pltpu.MemorySpace.VMEM),
        out_shape=jax.ShapeDtypeStruct((1, 1), jnp.float32),
        scratch_shapes=[
            pltpu.VMEM((NBUF, CH, C), jnp.float32),
            pltpu.VMEM((NCH, CH), jnp.float32),
            pltpu.SemaphoreType.DMA((NBUF,)),
        ],
    )(pred, tgt3)
    return out[0, 0]


# speedup vs baseline: 1.0651x; 1.0272x over previous
import jax
import jax.numpy as jnp
from jax.experimental import pallas as pl
from jax.experimental.pallas import tpu as pltpu

N = 16384
C = 1000
K = N // 2
CH = 1024
NCH = N // CH
NBUF = 4


def _body(pred_hbm, tgt_ref, out_ref, buf, loss_ref, sems):
    def start(i):
        pltpu.make_async_copy(
            pred_hbm.at[pl.ds(i * CH, CH), :], buf.at[i % NBUF],
            sems.at[i % NBUF]).start()

    for i in range(NBUF):
        start(i)

    for c in range(NCH):
        pltpu.make_async_copy(
            pred_hbm.at[pl.ds(c * CH, CH), :], buf.at[c % NBUF],
            sems.at[c % NBUF]).wait()
        x = buf[c % NBUF]
        t = tgt_ref[c, 0, :]
        m = jnp.max(x, axis=1, keepdims=True)
        s = jnp.sum(jnp.exp(x - m), axis=1, keepdims=True)
        lse = m + jnp.log(s)
        col = jax.lax.broadcasted_iota(jnp.int32, x.shape, 1)
        picked = jnp.sum(jnp.where(col == t[:, None], x, 0.0), axis=1,
                         keepdims=True)
        loss_ref[c, :] = (lse - picked)[:, 0]
        if c + NBUF < NCH:
            start(c + NBUF)

    x = loss_ref[...]
    bits = jax.lax.bitcast_convert_type(x, jnp.int32)

    def bsearch(_, carry):
        lo, hi = carry
        mid = lo + (hi - lo) // 2
        cnt = jnp.sum((bits >= mid).astype(jnp.int32))
        take = cnt >= K
        return jnp.where(take, mid, lo), jnp.where(take, hi, mid)

    lo, _ = jax.lax.fori_loop(0, 31, bsearch,
                              (jnp.int32(0), jnp.int32(0x7F800000)))
    thr = jax.lax.bitcast_convert_type(lo, jnp.float32)
    gt = bits > lo
    cnt_gt = jnp.sum(gt.astype(jnp.int32))
    sum_gt = jnp.sum(jnp.where(gt, x, 0.0))
    total = sum_gt + (K - cnt_gt).astype(jnp.float32) * thr
    out_ref[...] = jnp.reshape(total / jnp.float32(K), (1, 1))


@jax.jit
def kernel(pred, target):
    tgt3 = target.reshape(NCH, 1, CH)
    out = pl.pallas_call(
        _body,
        in_specs=[
            pl.BlockSpec(memory_space=pltpu.MemorySpace.HBM),
            pl.BlockSpec(memory_space=pltpu.MemorySpace.VMEM),
        ],
        out_specs=pl.BlockSpec(memory_space=pltpu.MemorySpace.VMEM),
        out_shape=jax.ShapeDtypeStruct((1, 1), jnp.float32),
        scratch_shapes=[
            pltpu.VMEM((NBUF, CH, C), jnp.float32),
            pltpu.VMEM((NCH, CH), jnp.float32),
            pltpu.SemaphoreType.DMA((NBUF,)),
        ],
    )(pred, tgt3)
    return out[0, 0]
